# baseline (speedup 1.0000x reference)
import dataclasses

import jax
import jax.numpy as jnp
from jax.experimental import pallas as pl
from jax.experimental.pallas import tpu as pltpu
from jax.experimental.pallas import tpu_sc as plsc

_LANES = 16
_NW = 32


def _round_bf16(v):
    b = jax.lax.bitcast_convert_type(v, jnp.uint32)
    r = (b + jnp.uint32(0x7FFF) + ((b >> 16) & jnp.uint32(1))) \
        & jnp.uint32(0xFFFF0000)
    return jax.lax.bitcast_convert_type(r, jnp.float32)


def _pass_dir(qv, cv, q_lo, n_own, n_cand, total):
    f32 = jnp.float32

    def q_chunk(qi, tot):
        qs = q_lo + qi * _LANES
        q0v = qv[0, pl.ds(qs, _LANES)]
        q1v = qv[1, pl.ds(qs, _LANES)]
        q2v = qv[2, pl.ds(qs, _LANES)]
        q0b = _round_bf16(q0v)
        q1b = _round_bf16(q1v)
        q2b = _round_bf16(q2v)
        qn = (q0v * q0v + q1v * q1v) + q2v * q2v

        def cand_chunk(j, acc):
            c0 = cv[0, pl.ds(j * _LANES, _LANES)]
            c1 = cv[1, pl.ds(j * _LANES, _LANES)]
            c2 = cv[2, pl.ds(j * _LANES, _LANES)]
            cb0 = _round_bf16(c0)
            cb1 = _round_bf16(c1)
            cb2 = _round_bf16(c2)
            cn = (c0 * c0 + c1 * c1) + c2 * c2
            for k in range(_LANES):
                xy = (q0b * cb0[k] + q1b * cb1[k]) + q2b * cb2[k]
                d = (qn + cn[k]) - 2.0 * xy
                acc = jnp.minimum(acc, d)
            return acc

        mins = jax.lax.fori_loop(
            0, n_cand // _LANES, cand_chunk,
            jnp.full((_LANES,), jnp.inf, f32))
        return tot + jnp.maximum(mins, 0.0)

    return jax.lax.fori_loop(0, n_own // _LANES, q_chunk, total)


def kernel(x, y):
    B, N, _ = x.shape
    M = y.shape[1]
    xt = jnp.swapaxes(x, 1, 2)
    yt = jnp.swapaxes(y, 1, 2)
    nx = N // _NW
    ny = M // _NW

    mesh = plsc.VectorSubcoreMesh(core_axis_name="c", subcore_axis_name="s")
    cp = pltpu.CompilerParams()
    if "needs_layout_passes" in pltpu.CompilerParams.__dataclass_fields__:
        cp = dataclasses.replace(cp, needs_layout_passes=False)

    @pl.kernel(
        out_type=jax.ShapeDtypeStruct((2, 16, 1, _LANES), jnp.float32),
        mesh=mesh,
        compiler_params=cp,
        scratch_types=[
            pltpu.VMEM((3, N), jnp.float32),
            pltpu.VMEM((3, M), jnp.float32),
            pltpu.VMEM((1, _LANES), jnp.float32),
            pltpu.SemaphoreType.DMA,
        ],
    )
    def sc_kernel(xt_hbm, yt_hbm, o_hbm, xv, yv, ov, sem):
        core = jax.lax.axis_index("c")
        sub = jax.lax.axis_index("s")
        w = core * 16 + sub

        def per_batch(b, tot):
            pltpu.async_copy(xt_hbm.at[b], xv, sem).wait()
            pltpu.async_copy(yt_hbm.at[b], yv, sem).wait()
            tot = _pass_dir(xv, yv, w * nx, nx, M, tot)
            tot = _pass_dir(yv, xv, w * ny, ny, N, tot)
            return tot

        total = jax.lax.fori_loop(
            0, B, per_batch, jnp.zeros((_LANES,), jnp.float32))
        ov[0, :] = total
        pltpu.async_copy(ov, o_hbm.at[core, sub], sem).wait()

    psum = sc_kernel(xt, yt)
    return jnp.sum(psum) / x.shape[0]


# speedup vs baseline: 20.2400x; 20.2400x over previous
import jax
import jax.numpy as jnp
from jax.experimental import pallas as pl
from jax.experimental.pallas import tpu as pltpu

_K = 16


def _bf16_split3(v):
    hi = v.astype(jnp.bfloat16)
    r = v - hi.astype(jnp.float32)
    mid = r.astype(jnp.bfloat16)
    lo = (r - mid.astype(jnp.float32)).astype(jnp.bfloat16)
    return hi, mid, lo


def _augment_t(pt, norm_first):
    P = pt.shape[1]
    bf = jnp.bfloat16
    nrm = (pt[0:1] * pt[0:1] + pt[1:2] * pt[1:2]) + pt[2:3] * pt[2:3]
    hi, mid, lo = _bf16_split3(nrm)
    ones = jnp.ones((3, P), bf)
    zeros = jnp.zeros((_K - 9, P), bf)
    if norm_first:
        pieces = [(-2.0 * pt).astype(bf), hi, mid, lo, ones, zeros]
    else:
        pieces = [pt.astype(bf), ones, hi, mid, lo, zeros]
    return jnp.concatenate(pieces, axis=0)


def _chamfer_body(xt_ref, yt_ref, loss_ref):
    nb = xt_ref.shape[0]
    total = jnp.float32(0.0)
    for b in range(nb):
        xat = _augment_t(xt_ref[b], True)
        ya = _augment_t(yt_ref[b], False)
        dist = jax.lax.dot_general(
            xat, ya, (((0,), (0,)), ((), ())),
            preferred_element_type=jnp.float32)
        row_min = jnp.maximum(jnp.min(dist, axis=1), 0.0)
        col_min = jnp.maximum(jnp.min(dist, axis=0), 0.0)
        total += jnp.sum(row_min) + jnp.sum(col_min)
    loss_ref[0, 0] = total * (1.0 / nb)


def kernel(x, y):
    B, N, _ = x.shape
    M = y.shape[1]
    xt = jnp.swapaxes(x, 1, 2)
    yt = jnp.swapaxes(y, 1, 2)

    loss = pl.pallas_call(
        _chamfer_body,
        grid=(1,),
        in_specs=[
            pl.BlockSpec((B, 3, N), lambda i: (0, 0, 0)),
            pl.BlockSpec((B, 3, M), lambda i: (0, 0, 0)),
        ],
        out_specs=pl.BlockSpec(
            (1, 1), lambda i: (0, 0), memory_space=pltpu.SMEM),
        out_shape=jax.ShapeDtypeStruct((1, 1), jnp.float32),
    )(xt, yt)
    return jnp.reshape(loss, ())
